# baseline (speedup 1.0000x reference)
import functools

import jax
import jax.numpy as jnp
from jax import lax
from jax.experimental import pallas as pl
from jax.experimental.pallas import tpu as pltpu
from jax.experimental.pallas import tpu_sc as plsc

_N_E = 8192
_E_DIM = 256
_BETA = 0.25
_M = 9216
_BM = 384
_MT = _M // _BM


def _vq_body(z_ref, e2_ref, idx_ref, oh_ref, loss_ref, ppl_ref,
             counts_acc, loss_acc, en_acc):
    i = pl.program_id(0)
    z = z_ref[...]
    zn = jnp.sum(z * z, axis=1, keepdims=True)

    @pl.when(i == 0)
    def _():
        e2c = e2_ref[...]
        en_acc[...] = 0.25 * jnp.sum(e2c * e2c, axis=1, keepdims=True).T

    en = en_acc[...]
    mm2 = lax.dot_general(z, e2_ref[...], (((1,), (1,)), ((), ())),
                          preferred_element_type=jnp.float32)
    d = (zn + en) + mm2
    dmin = jnp.min(d, axis=1, keepdims=True)
    col = lax.broadcasted_iota(jnp.int32, d.shape, 1)
    idx = jnp.min(jnp.where(d == dmin, col, _N_E), axis=1, keepdims=True)
    oh = (col == idx).astype(jnp.float32)
    oh_ref[...] = oh
    idx_ref[...] = idx

    part = lax.dot_general(jnp.ones((1, _BM), jnp.float32), oh,
                           (((1,), (0,)), ((), ())),
                           preferred_element_type=jnp.float32)
    psum = jnp.sum(dmin, axis=0, keepdims=True)

    @pl.when(i == 0)
    def _():
        counts_acc[...] = part
        loss_acc[...] = psum

    @pl.when(i > 0)
    def _():
        counts_acc[...] = counts_acc[...] + part
        loss_acc[...] = loss_acc[...] + psum

    @pl.when(i == _MT - 1)
    def _():
        loss_ref[...] = loss_acc[...] * ((1.0 + _BETA) / (_M * _E_DIM))
        e_mean = counts_acc[...] / jnp.float32(_M)
        ent = jnp.sum(e_mean * jnp.log(e_mean + 1e-10), axis=1, keepdims=True)
        ppl_ref[...] = jnp.exp(-ent)


_vq_call = pl.pallas_call(
    _vq_body,
    grid=(_MT,),
    in_specs=[
        pl.BlockSpec((_BM, _E_DIM), lambda i: (i, 0)),
        pl.BlockSpec((_N_E, _E_DIM), lambda i: (0, 0)),
    ],
    out_specs=[
        pl.BlockSpec((_BM, 1), lambda i: (i, 0)),
        pl.BlockSpec((_BM, _N_E), lambda i: (i, 0)),
        pl.BlockSpec((1, 1), lambda i: (0, 0)),
        pl.BlockSpec((1, 1), lambda i: (0, 0)),
    ],
    out_shape=[
        jax.ShapeDtypeStruct((_M, 1), jnp.int32),
        jax.ShapeDtypeStruct((_M, _N_E), jnp.float32),
        jax.ShapeDtypeStruct((1, 1), jnp.float32),
        jax.ShapeDtypeStruct((1, 1), jnp.float32),
    ],
    scratch_shapes=[
        pltpu.VMEM((1, _N_E), jnp.float32),
        pltpu.VMEM((1, 1), jnp.float32),
        pltpu.VMEM((1, _N_E), jnp.float32),
    ],
)


_NC = 2
_NS = 16
_NW = _NC * _NS
_BPW = _M // _NW


@functools.partial(
    pl.kernel,
    mesh=plsc.VectorSubcoreMesh(core_axis_name="c", subcore_axis_name="s"),
    out_type=jax.ShapeDtypeStruct((_M, _E_DIM), jnp.float32),
    scratch_types=[
        pltpu.VMEM((_BPW,), jnp.int32),
        pltpu.VMEM((_BPW, _E_DIM), jnp.float32),
        pltpu.SemaphoreType.DMA,
    ],
)
def _sc_gather(table_hbm, idx_hbm, out_hbm, idx_v, rows_v, sem):
    wid = lax.axis_index("s") * _NC + lax.axis_index("c")
    base = wid * _BPW
    pltpu.sync_copy(idx_hbm.at[pl.ds(base, _BPW)], idx_v)
    pltpu.async_copy(table_hbm.at[idx_v], rows_v, sem).wait()
    pltpu.sync_copy(rows_v, out_hbm.at[pl.ds(base, _BPW)])


def kernel(z, embedding_weight):
    zf = z.reshape(-1, _E_DIM)
    idx, oh, loss, ppl = _vq_call(zf, -2.0 * embedding_weight)
    zq = _sc_gather(embedding_weight, idx.reshape(_M))
    return (loss[0, 0], zq.reshape(z.shape), ppl[0, 0], oh, idx)


# speedup vs baseline: 1.0340x; 1.0340x over previous
import functools

import jax
import jax.numpy as jnp
from jax import lax
from jax.experimental import pallas as pl
from jax.experimental.pallas import tpu as pltpu
from jax.experimental.pallas import tpu_sc as plsc

_N_E = 8192
_E_DIM = 256
_BETA = 0.25
_M = 9216
_BM = 384
_MT = _M // _BM
_CW = 1024


def _vq_body(z_ref, e_ref, idx_ref, oh_ref, loss_ref, ppl_ref,
             counts_acc, loss_acc, en_acc):
    i = pl.program_id(0)
    z = z_ref[...]
    zn = jnp.sum(z * z, axis=1, keepdims=True)

    @pl.when(i == 0)
    def _():
        e = e_ref[...]
        en_acc[...] = jnp.sum(e * e, axis=1, keepdims=True).T

    en = en_acc[...]
    e = e_ref[...]

    dmin = None
    idx = None
    for c in range(_N_E // _CW):
        ec = e[c * _CW:(c + 1) * _CW, :]
        enc = en[:, c * _CW:(c + 1) * _CW]
        mm = lax.dot_general(z, ec, (((1,), (1,)), ((), ())),
                             preferred_element_type=jnp.float32)
        dc = (zn + enc) - 2.0 * mm
        mc = jnp.min(dc, axis=1, keepdims=True)
        colc = lax.broadcasted_iota(jnp.int32, dc.shape, 1) + c * _CW
        ic = jnp.min(jnp.where(dc == mc, colc, _N_E), axis=1, keepdims=True)
        if c == 0:
            dmin, idx = mc, ic
        else:
            idx = jnp.where(mc < dmin, ic, idx)
            dmin = jnp.minimum(mc, dmin)
    idx_ref[...] = idx

    for c in range(_N_E // _CW):
        colc = lax.broadcasted_iota(jnp.int32, (_BM, _CW), 1) + c * _CW
        oh_ref[:, c * _CW:(c + 1) * _CW] = (colc == idx).astype(jnp.float32)

    part = lax.dot_general(jnp.ones((1, _BM), jnp.float32), oh_ref[...],
                           (((1,), (0,)), ((), ())),
                           preferred_element_type=jnp.float32)
    psum = jnp.sum(dmin, axis=0, keepdims=True)

    @pl.when(i == 0)
    def _():
        counts_acc[...] = part
        loss_acc[...] = psum

    @pl.when(i > 0)
    def _():
        counts_acc[...] = counts_acc[...] + part
        loss_acc[...] = loss_acc[...] + psum

    @pl.when(i == _MT - 1)
    def _():
        loss_ref[...] = loss_acc[...] * ((1.0 + _BETA) / (_M * _E_DIM))
        e_mean = counts_acc[...] / jnp.float32(_M)
        ent = jnp.sum(e_mean * jnp.log(e_mean + 1e-10), axis=1, keepdims=True)
        ppl_ref[...] = jnp.exp(-ent)


_vq_call = pl.pallas_call(
    _vq_body,
    grid=(_MT,),
    in_specs=[
        pl.BlockSpec((_BM, _E_DIM), lambda i: (i, 0)),
        pl.BlockSpec((_N_E, _E_DIM), lambda i: (0, 0)),
    ],
    out_specs=[
        pl.BlockSpec((_BM, 1), lambda i: (i, 0)),
        pl.BlockSpec((_BM, _N_E), lambda i: (i, 0)),
        pl.BlockSpec((1, 1), lambda i: (0, 0)),
        pl.BlockSpec((1, 1), lambda i: (0, 0)),
    ],
    out_shape=[
        jax.ShapeDtypeStruct((_M, 1), jnp.int32),
        jax.ShapeDtypeStruct((_M, _N_E), jnp.float32),
        jax.ShapeDtypeStruct((1, 1), jnp.float32),
        jax.ShapeDtypeStruct((1, 1), jnp.float32),
    ],
    scratch_shapes=[
        pltpu.VMEM((1, _N_E), jnp.float32),
        pltpu.VMEM((1, 1), jnp.float32),
        pltpu.VMEM((1, _N_E), jnp.float32),
    ],
)


_NC = 2
_NS = 16
_NW = _NC * _NS
_BPW = _M // _NW


@functools.partial(
    pl.kernel,
    mesh=plsc.VectorSubcoreMesh(core_axis_name="c", subcore_axis_name="s"),
    out_type=jax.ShapeDtypeStruct((_M, _E_DIM), jnp.float32),
    scratch_types=[
        pltpu.VMEM((_BPW,), jnp.int32),
        pltpu.VMEM((_BPW, _E_DIM), jnp.float32),
        pltpu.SemaphoreType.DMA,
    ],
)
def _sc_gather(table_hbm, idx_hbm, out_hbm, idx_v, rows_v, sem):
    wid = lax.axis_index("s") * _NC + lax.axis_index("c")
    base = wid * _BPW
    pltpu.sync_copy(idx_hbm.at[pl.ds(base, _BPW)], idx_v)
    pltpu.async_copy(table_hbm.at[idx_v], rows_v, sem).wait()
    pltpu.sync_copy(rows_v, out_hbm.at[pl.ds(base, _BPW)])


def kernel(z, embedding_weight):
    zf = z.reshape(-1, _E_DIM)
    idx, oh, loss, ppl = _vq_call(zf, embedding_weight)
    zq = _sc_gather(embedding_weight, idx.reshape(_M))
    return (loss[0, 0], zq.reshape(z.shape), ppl[0, 0], oh, idx)


# speedup vs baseline: 1.1426x; 1.1050x over previous
import functools

import jax
import jax.numpy as jnp
from jax import lax
from jax.experimental import pallas as pl
from jax.experimental.pallas import tpu as pltpu
from jax.experimental.pallas import tpu_sc as plsc

_N_E = 8192
_E_DIM = 256
_BETA = 0.25
_M = 9216
_BM = 384
_MT = _M // _BM


def _en_body(e_ref, en_ref):
    e = e_ref[...]
    en_ref[...] = jnp.sum(e * e, axis=1, keepdims=True).T


_en_call = pl.pallas_call(
    _en_body,
    out_shape=jax.ShapeDtypeStruct((1, _N_E), jnp.float32),
)


def _vq_body(z_ref, e_ref, en_ref, idx_ref, oh_ref, loss_ref, ppl_ref,
             counts_acc, loss_acc):
    i = pl.program_id(0)
    z = z_ref[...]
    zn = jnp.sum(z * z, axis=1, keepdims=True)
    en = en_ref[...]
    mm = lax.dot_general(z, e_ref[...], (((1,), (1,)), ((), ())),
                         preferred_element_type=jnp.float32)
    d = (zn + en) - 2.0 * mm
    dmin = jnp.min(d, axis=1, keepdims=True)
    col = lax.broadcasted_iota(jnp.int32, d.shape, 1)
    idx = jnp.min(jnp.where(d == dmin, col, _N_E), axis=1, keepdims=True)
    oh = (col == idx).astype(jnp.float32)
    oh_ref[...] = oh
    idx_ref[...] = idx

    part = lax.dot_general(jnp.ones((1, _BM), jnp.float32), oh,
                           (((1,), (0,)), ((), ())),
                           preferred_element_type=jnp.float32)
    psum = jnp.sum(dmin, axis=0, keepdims=True)

    first = i == 0
    counts_acc[...] = part + jnp.where(first, 0.0, counts_acc[...])
    loss_acc[...] = psum + jnp.where(first, 0.0, loss_acc[...])

    @pl.when(i == _MT - 1)
    def _():
        loss_ref[...] = loss_acc[...] * ((1.0 + _BETA) / (_M * _E_DIM))
        e_mean = counts_acc[...] / jnp.float32(_M)
        ent = jnp.sum(e_mean * jnp.log(e_mean + 1e-10), axis=1, keepdims=True)
        ppl_ref[...] = jnp.exp(-ent)


_vq_call = pl.pallas_call(
    _vq_body,
    grid=(_MT,),
    in_specs=[
        pl.BlockSpec((_BM, _E_DIM), lambda i: (i, 0)),
        pl.BlockSpec((_N_E, _E_DIM), lambda i: (0, 0)),
        pl.BlockSpec((1, _N_E), lambda i: (0, 0)),
    ],
    out_specs=[
        pl.BlockSpec((_BM, 1), lambda i: (i, 0)),
        pl.BlockSpec((_BM, _N_E), lambda i: (i, 0)),
        pl.BlockSpec((1, 1), lambda i: (0, 0)),
        pl.BlockSpec((1, 1), lambda i: (0, 0)),
    ],
    out_shape=[
        jax.ShapeDtypeStruct((_M, 1), jnp.int32),
        jax.ShapeDtypeStruct((_M, _N_E), jnp.float32),
        jax.ShapeDtypeStruct((1, 1), jnp.float32),
        jax.ShapeDtypeStruct((1, 1), jnp.float32),
    ],
    scratch_shapes=[
        pltpu.VMEM((1, _N_E), jnp.float32),
        pltpu.VMEM((1, 1), jnp.float32),
    ],
)


_NC = 2
_NS = 16
_NW = _NC * _NS
_BPW = _M // _NW


@functools.partial(
    pl.kernel,
    mesh=plsc.VectorSubcoreMesh(core_axis_name="c", subcore_axis_name="s"),
    out_type=jax.ShapeDtypeStruct((_M, _E_DIM), jnp.float32),
    scratch_types=[
        pltpu.VMEM((_BPW,), jnp.int32),
        pltpu.VMEM((_BPW, _E_DIM), jnp.float32),
        pltpu.SemaphoreType.DMA,
    ],
)
def _sc_gather(table_hbm, idx_hbm, out_hbm, idx_v, rows_v, sem):
    wid = lax.axis_index("s") * _NC + lax.axis_index("c")
    base = wid * _BPW
    pltpu.sync_copy(idx_hbm.at[pl.ds(base, _BPW)], idx_v)
    pltpu.async_copy(table_hbm.at[idx_v], rows_v, sem).wait()
    pltpu.sync_copy(rows_v, out_hbm.at[pl.ds(base, _BPW)])


def kernel(z, embedding_weight):
    zf = z.reshape(-1, _E_DIM)
    en = _en_call(embedding_weight)
    idx, oh, loss, ppl = _vq_call(zf, embedding_weight, en)
    zq = _sc_gather(embedding_weight, idx.reshape(_M))
    return (loss[0, 0], zq.reshape(z.shape), ppl[0, 0], oh, idx)


# speedup vs baseline: 1.1467x; 1.0036x over previous
import functools

import jax
import jax.numpy as jnp
from jax import lax
from jax.experimental import pallas as pl
from jax.experimental.pallas import tpu as pltpu
from jax.experimental.pallas import tpu_sc as plsc

_N_E = 8192
_E_DIM = 256
_BETA = 0.25
_M = 9216
_BM = 384
_MT = _M // _BM


def _en_body(e_ref, en_ref):
    e = e_ref[...]
    en_ref[...] = jnp.sum(e * e, axis=1, keepdims=True).T


_en_call = pl.pallas_call(
    _en_body,
    out_shape=jax.ShapeDtypeStruct((1, _N_E), jnp.float32),
)


def _vq_body(z_ref, e_ref, en_ref, idx_ref, oh_ref, counts_ref, lsum_ref):
    i = pl.program_id(0)
    z = z_ref[...]
    zn = jnp.sum(z * z, axis=1, keepdims=True)
    en = en_ref[...]
    mm = lax.dot_general(z, e_ref[...], (((1,), (1,)), ((), ())),
                         preferred_element_type=jnp.float32)
    d = (zn + en) - 2.0 * mm
    dmin = jnp.min(d, axis=1, keepdims=True)
    col = lax.broadcasted_iota(jnp.int32, d.shape, 1)
    idx = jnp.min(jnp.where(d == dmin, col, _N_E), axis=1, keepdims=True)
    oh = (col == idx).astype(jnp.float32)
    oh_ref[...] = oh
    idx_ref[...] = idx

    part = lax.dot_general(jnp.ones((1, _BM), jnp.float32), oh,
                           (((1,), (0,)), ((), ())),
                           preferred_element_type=jnp.float32)
    psum = jnp.sum(dmin, axis=0, keepdims=True)

    first = i == 0
    counts_ref[...] = part + jnp.where(first, 0.0, counts_ref[...])
    lsum_ref[...] = psum + jnp.where(first, 0.0, lsum_ref[...])


_vq_call = pl.pallas_call(
    _vq_body,
    grid=(_MT,),
    in_specs=[
        pl.BlockSpec((_BM, _E_DIM), lambda i: (i, 0)),
        pl.BlockSpec((_N_E, _E_DIM), lambda i: (0, 0)),
        pl.BlockSpec((1, _N_E), lambda i: (0, 0)),
    ],
    out_specs=[
        pl.BlockSpec((_BM, 1), lambda i: (i, 0)),
        pl.BlockSpec((_BM, _N_E), lambda i: (i, 0)),
        pl.BlockSpec((1, _N_E), lambda i: (0, 0)),
        pl.BlockSpec((1, 1), lambda i: (0, 0)),
    ],
    out_shape=[
        jax.ShapeDtypeStruct((_M, 1), jnp.int32),
        jax.ShapeDtypeStruct((_M, _N_E), jnp.float32),
        jax.ShapeDtypeStruct((1, _N_E), jnp.float32),
        jax.ShapeDtypeStruct((1, 1), jnp.float32),
    ],
)


def _fin_body(counts_ref, lsum_ref, loss_ref, ppl_ref):
    loss_ref[...] = lsum_ref[...] * ((1.0 + _BETA) / (_M * _E_DIM))
    e_mean = counts_ref[...] / jnp.float32(_M)
    ent = jnp.sum(e_mean * jnp.log(e_mean + 1e-10), axis=1, keepdims=True)
    ppl_ref[...] = jnp.exp(-ent)


_fin_call = pl.pallas_call(
    _fin_body,
    out_shape=[
        jax.ShapeDtypeStruct((1, 1), jnp.float32),
        jax.ShapeDtypeStruct((1, 1), jnp.float32),
    ],
)


_NC = 2
_NS = 16
_NW = _NC * _NS
_BPW = _M // _NW


@functools.partial(
    pl.kernel,
    mesh=plsc.VectorSubcoreMesh(core_axis_name="c", subcore_axis_name="s"),
    out_type=jax.ShapeDtypeStruct((_M, _E_DIM), jnp.float32),
    scratch_types=[
        pltpu.VMEM((_BPW,), jnp.int32),
        pltpu.VMEM((_BPW, _E_DIM), jnp.float32),
        pltpu.SemaphoreType.DMA,
    ],
)
def _sc_gather(table_hbm, idx_hbm, out_hbm, idx_v, rows_v, sem):
    wid = lax.axis_index("s") * _NC + lax.axis_index("c")
    base = wid * _BPW
    pltpu.sync_copy(idx_hbm.at[pl.ds(base, _BPW)], idx_v)
    pltpu.async_copy(table_hbm.at[idx_v], rows_v, sem).wait()
    pltpu.sync_copy(rows_v, out_hbm.at[pl.ds(base, _BPW)])


def kernel(z, embedding_weight):
    zf = z.reshape(-1, _E_DIM)
    en = _en_call(embedding_weight)
    idx, oh, counts, lsum = _vq_call(zf, embedding_weight, en)
    loss, ppl = _fin_call(counts, lsum)
    zq = _sc_gather(embedding_weight, idx.reshape(_M))
    return (loss[0, 0], zq.reshape(z.shape), ppl[0, 0], oh, idx)
